# baseline (speedup 1.0000x reference)
import functools

import jax
import jax.numpy as jnp
from jax import lax
from jax.experimental import pallas as pl
from jax.experimental.pallas import tpu as pltpu
from jax.experimental.pallas import tpu_sc as plsc

_N, _E, _D, _G = 10000, 320000, 128, 256
_NC, _NS = 2, 16
_NW = _NC * _NS
_K = 128
_CH = 80
_EP = _NW * _CH * _K
_NP = 10240
_RT = _NP // _NS
_RB = 2048
_RM = 1024
_EPS = 1e-5
_DW = 128

_f32 = jnp.float32
_HI = lax.Precision.HIGHEST


def _sc_mesh():
    return plsc.VectorSubcoreMesh(
        core_axis_name="c", subcore_axis_name="s",
        num_cores=_NC, num_subcores=_NS)


@functools.cache
def _build_sc_degree():
    return functools.partial(
        pl.kernel,
        out_type=jax.ShapeDtypeStruct((_NC, _NP, _DW), _f32),
        mesh=_sc_mesh(),
        scratch_types=[
            pltpu.VMEM((_CH, _K), jnp.int32),
            pltpu.VMEM((_K, _DW), _f32),
            pltpu.VMEM_SHARED((_NP, _DW), _f32),
        ],
    )(_sc_degree_body)


def _sc_degree_body(col_hbm, ones_hbm, zeros8_hbm, out_hbm, col_v, ones_v, acc):
    cid = lax.axis_index("c")
    sid = lax.axis_index("s")
    w = cid * _NS + sid
    pltpu.sync_copy(zeros8_hbm.at[pl.ds(sid * _RT, _RT)],
                    acc.at[pl.ds(sid * _RT, _RT)])
    pltpu.sync_copy(col_hbm.at[w], col_v)
    pltpu.sync_copy(ones_hbm, ones_v)
    plsc.subcore_barrier()

    def chunk(j, carry):
        pltpu.sync_copy(ones_v, acc.at[col_v.at[j]], add=True)
        return carry

    lax.fori_loop(0, _CH, chunk, 0)
    plsc.subcore_barrier()
    pltpu.sync_copy(acc.at[pl.ds(sid * _RT, _RT)],
                    out_hbm.at[cid, pl.ds(sid * _RT, _RT)])


def _sc_degree(colp, ones8, zeros8):
    return _build_sc_degree()(colp, ones8, zeros8)


@functools.cache
def _build_sc_scatter():
    return functools.partial(
        pl.kernel,
        out_type=jax.ShapeDtypeStruct((_NC, _NP, _D), _f32),
        mesh=_sc_mesh(),
        scratch_types=[
            pltpu.VMEM((2, _K), jnp.int32),
            pltpu.VMEM((_CH, _K), jnp.int32),
            pltpu.VMEM((_K, _D), _f32),
            pltpu.VMEM((_K, _D), _f32),
            pltpu.VMEM_SHARED((_NP, _D), _f32),
            pltpu.SemaphoreType.DMA,
            pltpu.SemaphoreType.DMA,
            pltpu.SemaphoreType.DMA,
            pltpu.SemaphoreType.DMA,
        ],
    )(_sc_scatter_body)


_NSUB = 4
_KS = _K // _NSUB


def _sc_scatter_body(u_hbm, row_hbm, col_hbm, zeros_hbm, out_hbm,
                     rv, col_v, buf0, buf1, acc, sg0, sg1, sr0, sr1):
    cid = lax.axis_index("c")
    sid = lax.axis_index("s")
    w = cid * _NS + sid
    pltpu.sync_copy(zeros_hbm.at[pl.ds(sid * _RT, _RT)],
                    acc.at[pl.ds(sid * _RT, _RT)])
    pltpu.sync_copy(col_hbm.at[w], col_v)
    bufs = (buf0, buf1)
    sgs = (sg0, sg1)
    srs = (sr0, sr1)

    def gathers(slot, buf, sg):
        for q in range(_NSUB):
            pltpu.async_copy(u_hbm.at[rv.at[slot, pl.ds(q * _KS, _KS)]],
                             buf.at[pl.ds(q * _KS, _KS)], sg)

    def drain(slot, buf, sg):
        for q in range(_NSUB):
            pltpu.make_async_copy(
                u_hbm.at[rv.at[slot, pl.ds(q * _KS, _KS)]],
                buf.at[pl.ds(q * _KS, _KS)], sg).wait()

    pltpu.sync_copy(row_hbm.at[w, 0], rv.at[0])
    plsc.subcore_barrier()
    gathers(0, buf0, sg0)
    pltpu.async_copy(row_hbm.at[w, 1], rv.at[1], sr1)

    def step(j2, carry):
        for b in range(2):
            j = j2 * 2 + b
            drain(b, bufs[b], sgs[b])

            @pl.when(j + 1 < _CH)
            def _():
                pltpu.make_async_copy(row_hbm.at[w, j + 1], rv.at[1 - b],
                                      srs[1 - b]).wait()
                gathers(1 - b, bufs[1 - b], sgs[1 - b])

            @pl.when(j + 2 < _CH)
            def _():
                pltpu.async_copy(row_hbm.at[w, j + 2], rv.at[b], srs[b])

            pltpu.sync_copy(bufs[b], acc.at[col_v.at[j]], add=True)
        return carry

    lax.fori_loop(0, _CH // 2, step, 0)
    plsc.subcore_barrier()
    pltpu.sync_copy(acc.at[pl.ds(sid * _RT, _RT)],
                    out_hbm.at[cid, pl.ds(sid * _RT, _RT)])


def _sc_scatter(u, rowp, colp, zerosnd):
    return _build_sc_scatter()(u, rowp, colp, zerosnd)


def _tc_prep(deg8, x, w1):
    def body(deg8_ref, x_ref, w_ref, dis_ref, u_ref):
        d8 = deg8_ref[0] + deg8_ref[1]
        deg = d8[:, 0:1] + 1.0
        dis = lax.rsqrt(deg)
        dis_ref[...] = dis
        lin = jnp.dot(x_ref[...], w_ref[...],
                      preferred_element_type=_f32, precision=_HI)
        u_ref[pl.ds(0, _N), :] = dis[:_N] * lin
        u_ref[pl.ds(_N, _NP - _N), :] = jnp.zeros((_NP - _N, _D), _f32)

    return pl.pallas_call(
        body,
        out_shape=[jax.ShapeDtypeStruct((_NP, 1), _f32),
                   jax.ShapeDtypeStruct((_NP, _D), _f32)],
    )(deg8, x, w1)


def _bn_relu(acc_ref, u_ref, dis_ref, b_ref, g_ref, be_ref):
    a = acc_ref[...]
    s = a[0, :_N] + a[1, :_N] + u_ref[pl.ds(0, _N), :]
    dis = dis_ref[...][:_N]
    pre = dis * s + b_ref[...]
    mean = jnp.mean(pre, axis=0, keepdims=True)
    xc = pre - mean
    var = jnp.mean(xc * xc, axis=0, keepdims=True)
    return jnp.maximum(xc * lax.rsqrt(var + _EPS) * g_ref[...] + be_ref[...],
                       0.0)


def _tc_post(acc, u, dis, b, g, be, wn):
    def body(acc_ref, u_ref, dis_ref, b_ref, g_ref, be_ref, w_ref, out_ref):
        h = _bn_relu(acc_ref, u_ref, dis_ref, b_ref, g_ref, be_ref)
        nxt = jnp.dot(h, w_ref[...], preferred_element_type=_f32,
                      precision=_HI)
        out_ref[pl.ds(0, _N), :] = dis_ref[...][:_N] * nxt
        out_ref[pl.ds(_N, _NP - _N), :] = jnp.zeros((_NP - _N, _D), _f32)

    return pl.pallas_call(
        body, out_shape=jax.ShapeDtypeStruct((_NP, _D), _f32),
    )(acc, u, dis, b, g, be, wn)


def _tc_post_last(acc, u, dis, b, g, be):
    def body(acc_ref, u_ref, dis_ref, b_ref, g_ref, be_ref, out_ref):
        h = _bn_relu(acc_ref, u_ref, dis_ref, b_ref, g_ref, be_ref)
        out_ref[pl.ds(0, _N), :] = h
        out_ref[pl.ds(_N, _NP - _N), :] = jnp.zeros((_NP - _N, _D), _f32)

    return pl.pallas_call(
        body, out_shape=jax.ShapeDtypeStruct((_NP, _D), _f32),
    )(acc, u, dis, b, g, be)


def _tc_pool_sums(h3p, batchp):
    nb = _NP // _RB

    def body(h_ref, b_ref, sums_ref, cnt_ref):
        i = pl.program_id(0)
        seg = lax.broadcasted_iota(jnp.int32, (_G, _RB), 0)
        oh = (b_ref[...] == seg).astype(_f32)
        ps = jnp.dot(oh, h_ref[...], preferred_element_type=_f32,
                     precision=_HI)
        pc = jnp.broadcast_to(jnp.sum(oh, axis=1, keepdims=True), (_G, _D))

        @pl.when(i == 0)
        def _():
            sums_ref[...] = ps
            cnt_ref[...] = pc

        @pl.when(i > 0)
        def _():
            sums_ref[...] = sums_ref[...] + ps
            cnt_ref[...] = cnt_ref[...] + pc

    return pl.pallas_call(
        body,
        grid=(nb,),
        in_specs=[pl.BlockSpec((_RB, _D), lambda i: (i, 0)),
                  pl.BlockSpec((1, _RB), lambda i: (0, i))],
        out_specs=[pl.BlockSpec((_G, _D), lambda i: (0, 0)),
                   pl.BlockSpec((_G, _D), lambda i: (0, 0))],
        out_shape=[jax.ShapeDtypeStruct((_G, _D), _f32),
                   jax.ShapeDtypeStruct((_G, _D), _f32)],
    )(h3p, batchp)


def _tc_pool_max(start, end, h3p):
    def body(start_ref, end_ref, h_ref, out_ref):
        g = pl.program_id(0)
        s = start_ref[g]
        e = end_ref[g]
        b0 = lax.div(s, _RM)
        b1 = lax.div(e + (_RM - 1), _RM)

        def blk(b, acc):
            rows = lax.broadcasted_iota(jnp.int32, (_RM, 1), 0) + b * _RM
            m = jnp.logical_and(rows >= s, rows < e)
            vals = jnp.where(m, h_ref[pl.ds(b * _RM, _RM), :], 0.0)
            return jnp.maximum(acc, jnp.max(vals, axis=0, keepdims=True))

        acc = lax.fori_loop(b0, b1, blk, jnp.zeros((1, _D), _f32))
        out_ref[...] = acc.reshape(1, 1, _D)

    return pl.pallas_call(
        body,
        grid=(_G,),
        in_specs=[pl.BlockSpec(memory_space=pltpu.SMEM),
                  pl.BlockSpec(memory_space=pltpu.SMEM),
                  pl.BlockSpec(memory_space=pltpu.VMEM)],
        out_specs=pl.BlockSpec((1, 1, _D), lambda g: (g, 0, 0)),
        out_shape=jax.ShapeDtypeStruct((_G, 1, _D), _f32),
    )(start, end, h3p)


def _tc_mlp(sums, cnt, xmax, wf1, bf1, wf2, bf2, wf3, bf3):
    def body(sums_ref, cnt_ref, xmax_ref, w1_ref, b1_ref, w2_ref, b2_ref,
             w3_ref, b3_ref, out_ref):
        counts = cnt_ref[...][:, 0:1]
        mean = sums_ref[...] / jnp.maximum(counts, 1.0)
        z = jnp.concatenate([mean, xmax_ref[...].reshape(_G, _D)], axis=1)
        z = jnp.maximum(jnp.dot(z, w1_ref[...], preferred_element_type=_f32,
                                precision=_HI) + b1_ref[...], 0.0)
        z = jnp.maximum(jnp.dot(z, w2_ref[...], preferred_element_type=_f32,
                                precision=_HI) + b2_ref[...], 0.0)
        out_ref[...] = jnp.dot(z, w3_ref[...], preferred_element_type=_f32,
                               precision=_HI) + b3_ref[...]

    return pl.pallas_call(
        body, out_shape=jax.ShapeDtypeStruct((_G, 2), _f32),
    )(sums, cnt, xmax, wf1, bf1, wf2, bf2, wf3, bf3)


def kernel(x, edge_index, batch, Wc1, bc1, g1, be1, Wc2, bc2, g2, be2,
           Wc3, bc3, g3, be3, Wf1, bf1, Wf2, bf2, Wf3, bf3):
    ei = edge_index.astype(jnp.int32)
    row_s, col_s = lax.sort([ei[0], ei[1]], num_keys=1)
    padi = jnp.full((_EP - _E,), _NP - 1, jnp.int32)
    rowp = jnp.concatenate([row_s, padi]).reshape(_NW, _CH, _K)
    colp = jnp.concatenate([col_s, padi]).reshape(_NW, _CH, _K)

    zeros8 = jnp.zeros((_NP, _DW), _f32)
    ones8 = jnp.ones((_K, _DW), _f32)
    zerosnd = jnp.zeros((_NP, _D), _f32)

    deg8 = _sc_degree(colp, ones8, zeros8)
    dis, u1 = _tc_prep(deg8, x, Wc1)

    acc1 = _sc_scatter(u1, rowp, colp, zerosnd)
    u2 = _tc_post(acc1, u1, dis, bc1.reshape(1, _D), g1.reshape(1, _D),
                  be1.reshape(1, _D), Wc2)
    acc2 = _sc_scatter(u2, rowp, colp, zerosnd)
    u3 = _tc_post(acc2, u2, dis, bc2.reshape(1, _D), g2.reshape(1, _D),
                  be2.reshape(1, _D), Wc3)
    acc3 = _sc_scatter(u3, rowp, colp, zerosnd)
    h3 = _tc_post_last(acc3, u3, dis, bc3.reshape(1, _D), g3.reshape(1, _D),
                       be3.reshape(1, _D))

    batchp = jnp.concatenate(
        [batch.astype(jnp.int32), jnp.full((_NP - _N,), _G, jnp.int32)]
    ).reshape(1, _NP)
    sums, cnt = _tc_pool_sums(h3, batchp)
    c = cnt[:, 0]
    end = jnp.cumsum(c).astype(jnp.int32)
    start = end - c.astype(jnp.int32)
    xmax = _tc_pool_max(start, end, h3)

    return _tc_mlp(sums, cnt, xmax, Wf1, bf1.reshape(1, _D), Wf2,
                   bf2.reshape(1, _D // 2), Wf3, bf3.reshape(1, 2))


# speedup vs baseline: 1.4505x; 1.4505x over previous
import functools

import jax
import jax.numpy as jnp
from jax import lax
from jax.experimental import pallas as pl
from jax.experimental.pallas import tpu as pltpu
from jax.experimental.pallas import tpu_sc as plsc

_N, _E, _D, _G = 10000, 320000, 128, 256
_NC, _NS = 2, 16
_NW = _NC * _NS
_K = 128
_CH = 80
_EP = _NW * _CH * _K
_NP = 10240
_RT = _NP // _NS
_RB = 2048
_RM = 1024
_EPS = 1e-5
_DW = 128

_f32 = jnp.float32
_HI = lax.Precision.HIGHEST


def _sc_mesh():
    return plsc.VectorSubcoreMesh(
        core_axis_name="c", subcore_axis_name="s",
        num_cores=_NC, num_subcores=_NS)


@functools.cache
def _build_sc_degree():
    return functools.partial(
        pl.kernel,
        out_type=jax.ShapeDtypeStruct((_NC, _NP, _DW), _f32),
        mesh=_sc_mesh(),
        scratch_types=[
            pltpu.VMEM((_CH, _K), jnp.int32),
            pltpu.VMEM((_K, _DW), _f32),
            pltpu.VMEM_SHARED((_NP, _DW), _f32),
        ],
    )(_sc_degree_body)


def _sc_degree_body(col_hbm, ones_hbm, zeros8_hbm, out_hbm, col_v, ones_v, acc):
    cid = lax.axis_index("c")
    sid = lax.axis_index("s")
    w = cid * _NS + sid
    pltpu.sync_copy(zeros8_hbm.at[pl.ds(sid * _RT, _RT)],
                    acc.at[pl.ds(sid * _RT, _RT)])
    pltpu.sync_copy(col_hbm.at[w], col_v)
    pltpu.sync_copy(ones_hbm, ones_v)
    plsc.subcore_barrier()

    def chunk(j, carry):
        pltpu.sync_copy(ones_v, acc.at[col_v.at[j]], add=True)
        return carry

    lax.fori_loop(0, _CH, chunk, 0)
    plsc.subcore_barrier()
    pltpu.sync_copy(acc.at[pl.ds(sid * _RT, _RT)],
                    out_hbm.at[cid, pl.ds(sid * _RT, _RT)])


def _sc_degree(colp, ones8, zeros8):
    return _build_sc_degree()(colp, ones8, zeros8)


@functools.cache
def _build_sc_scatter():
    return functools.partial(
        pl.kernel,
        out_type=jax.ShapeDtypeStruct((_NC, _NP, _D), _f32),
        mesh=_sc_mesh(),
        scratch_types=[
            pltpu.VMEM((2, _K), jnp.int32),
            pltpu.VMEM((_CH, _K), jnp.int32),
            pltpu.VMEM((_K, _D), _f32),
            pltpu.VMEM((_K, _D), _f32),
            pltpu.VMEM_SHARED((_NP, _D), _f32),
            pltpu.SemaphoreType.DMA,
            pltpu.SemaphoreType.DMA,
            pltpu.SemaphoreType.DMA,
            pltpu.SemaphoreType.DMA,
        ],
    )(_sc_scatter_body)


_NSUB = 4
_KS = _K // _NSUB


def _sc_scatter_body(u_hbm, row_hbm, col_hbm, zeros_hbm, out_hbm,
                     rv, col_v, buf0, buf1, acc, sg0, sg1, sr0, sr1):
    cid = lax.axis_index("c")
    sid = lax.axis_index("s")
    w = cid * _NS + sid
    pltpu.sync_copy(zeros_hbm.at[pl.ds(sid * _RT, _RT)],
                    acc.at[pl.ds(sid * _RT, _RT)])
    pltpu.sync_copy(col_hbm.at[w], col_v)
    bufs = (buf0, buf1)
    sgs = (sg0, sg1)
    srs = (sr0, sr1)

    def gathers(slot, buf, sg):
        for q in range(_NSUB):
            pltpu.async_copy(u_hbm.at[rv.at[slot, pl.ds(q * _KS, _KS)]],
                             buf.at[pl.ds(q * _KS, _KS)], sg)

    def drain(slot, buf, sg):
        for q in range(_NSUB):
            pltpu.make_async_copy(
                u_hbm.at[rv.at[slot, pl.ds(q * _KS, _KS)]],
                buf.at[pl.ds(q * _KS, _KS)], sg).wait()

    pltpu.sync_copy(row_hbm.at[w, 0], rv.at[0])
    plsc.subcore_barrier()
    gathers(0, buf0, sg0)
    pltpu.async_copy(row_hbm.at[w, 1], rv.at[1], sr1)

    def step(j2, carry):
        for b in range(2):
            j = j2 * 2 + b
            drain(b, bufs[b], sgs[b])

            @pl.when(j + 1 < _CH)
            def _():
                pltpu.make_async_copy(row_hbm.at[w, j + 1], rv.at[1 - b],
                                      srs[1 - b]).wait()
                gathers(1 - b, bufs[1 - b], sgs[1 - b])

            @pl.when(j + 2 < _CH)
            def _():
                pltpu.async_copy(row_hbm.at[w, j + 2], rv.at[b], srs[b])

            pltpu.sync_copy(bufs[b], acc.at[col_v.at[j]], add=True)
        return carry

    lax.fori_loop(0, _CH // 2, step, 0)
    plsc.subcore_barrier()
    pltpu.sync_copy(acc.at[pl.ds(sid * _RT, _RT)],
                    out_hbm.at[cid, pl.ds(sid * _RT, _RT)])


def _sc_scatter(u, rowp, colp, zerosnd):
    return _build_sc_scatter()(u, rowp, colp, zerosnd)


def _tc_prep(deg8, x, w1):
    def body(deg8_ref, x_ref, w_ref, dis_ref, u_ref):
        d8 = deg8_ref[0] + deg8_ref[1]
        deg = d8[:, 0:1] + 1.0
        dis = lax.rsqrt(deg)
        dis_ref[...] = dis
        lin = jnp.dot(x_ref[...], w_ref[...],
                      preferred_element_type=_f32, precision=_HI)
        u_ref[pl.ds(0, _N), :] = dis[:_N] * lin
        u_ref[pl.ds(_N, _NP - _N), :] = jnp.zeros((_NP - _N, _D), _f32)

    return pl.pallas_call(
        body,
        out_shape=[jax.ShapeDtypeStruct((_NP, 1), _f32),
                   jax.ShapeDtypeStruct((_NP, _D), _f32)],
    )(deg8, x, w1)


def _bn_relu(acc_ref, u_ref, dis_ref, b_ref, g_ref, be_ref):
    a = acc_ref[...]
    s = a[0, :_N] + a[1, :_N] + u_ref[pl.ds(0, _N), :]
    dis = dis_ref[...][:_N]
    pre = dis * s + b_ref[...]
    mean = jnp.mean(pre, axis=0, keepdims=True)
    xc = pre - mean
    var = jnp.mean(xc * xc, axis=0, keepdims=True)
    return jnp.maximum(xc * lax.rsqrt(var + _EPS) * g_ref[...] + be_ref[...],
                       0.0)


def _tc_post(acc, u, dis, b, g, be, wn):
    def body(acc_ref, u_ref, dis_ref, b_ref, g_ref, be_ref, w_ref, out_ref):
        h = _bn_relu(acc_ref, u_ref, dis_ref, b_ref, g_ref, be_ref)
        nxt = jnp.dot(h, w_ref[...], preferred_element_type=_f32,
                      precision=_HI)
        out_ref[pl.ds(0, _N), :] = dis_ref[...][:_N] * nxt
        out_ref[pl.ds(_N, _NP - _N), :] = jnp.zeros((_NP - _N, _D), _f32)

    return pl.pallas_call(
        body, out_shape=jax.ShapeDtypeStruct((_NP, _D), _f32),
    )(acc, u, dis, b, g, be, wn)


def _tc_post_last(acc, u, dis, b, g, be):
    def body(acc_ref, u_ref, dis_ref, b_ref, g_ref, be_ref, out_ref):
        h = _bn_relu(acc_ref, u_ref, dis_ref, b_ref, g_ref, be_ref)
        out_ref[pl.ds(0, _N), :] = h
        out_ref[pl.ds(_N, _NP - _N), :] = jnp.zeros((_NP - _N, _D), _f32)

    return pl.pallas_call(
        body, out_shape=jax.ShapeDtypeStruct((_NP, _D), _f32),
    )(acc, u, dis, b, g, be)


def _tc_pool_sums(h3p, batchp):
    nb = _NP // _RB

    def body(h_ref, b_ref, sums_ref, cnt_ref):
        i = pl.program_id(0)
        seg = lax.broadcasted_iota(jnp.int32, (_G, _RB), 0)
        oh = (b_ref[...] == seg).astype(_f32)
        ps = jnp.dot(oh, h_ref[...], preferred_element_type=_f32,
                     precision=_HI)
        pc = jnp.broadcast_to(jnp.sum(oh, axis=1, keepdims=True), (_G, _D))

        @pl.when(i == 0)
        def _():
            sums_ref[...] = ps
            cnt_ref[...] = pc

        @pl.when(i > 0)
        def _():
            sums_ref[...] = sums_ref[...] + ps
            cnt_ref[...] = cnt_ref[...] + pc

    return pl.pallas_call(
        body,
        grid=(nb,),
        in_specs=[pl.BlockSpec((_RB, _D), lambda i: (i, 0)),
                  pl.BlockSpec((1, _RB), lambda i: (0, i))],
        out_specs=[pl.BlockSpec((_G, _D), lambda i: (0, 0)),
                   pl.BlockSpec((_G, _D), lambda i: (0, 0))],
        out_shape=[jax.ShapeDtypeStruct((_G, _D), _f32),
                   jax.ShapeDtypeStruct((_G, _D), _f32)],
    )(h3p, batchp)


def _tc_pool_max(start, end, h3p):
    def body(start_ref, end_ref, h_ref, out_ref):
        g = pl.program_id(0)
        s = start_ref[g]
        e = end_ref[g]
        b0 = lax.div(s, _RM)
        b1 = lax.div(e + (_RM - 1), _RM)

        def blk(b, acc):
            rows = lax.broadcasted_iota(jnp.int32, (_RM, 1), 0) + b * _RM
            m = jnp.logical_and(rows >= s, rows < e)
            vals = jnp.where(m, h_ref[pl.ds(b * _RM, _RM), :], 0.0)
            return jnp.maximum(acc, jnp.max(vals, axis=0, keepdims=True))

        acc = lax.fori_loop(b0, b1, blk, jnp.zeros((1, _D), _f32))
        out_ref[...] = acc.reshape(1, 1, _D)

    return pl.pallas_call(
        body,
        grid=(_G,),
        in_specs=[pl.BlockSpec(memory_space=pltpu.SMEM),
                  pl.BlockSpec(memory_space=pltpu.SMEM),
                  pl.BlockSpec(memory_space=pltpu.VMEM)],
        out_specs=pl.BlockSpec((1, 1, _D), lambda g: (g, 0, 0)),
        out_shape=jax.ShapeDtypeStruct((_G, 1, _D), _f32),
    )(start, end, h3p)


def _tc_mlp(sums, cnt, xmax, wf1, bf1, wf2, bf2, wf3, bf3):
    def body(sums_ref, cnt_ref, xmax_ref, w1_ref, b1_ref, w2_ref, b2_ref,
             w3_ref, b3_ref, out_ref):
        counts = cnt_ref[...][:, 0:1]
        mean = sums_ref[...] / jnp.maximum(counts, 1.0)
        z = jnp.concatenate([mean, xmax_ref[...].reshape(_G, _D)], axis=1)
        z = jnp.maximum(jnp.dot(z, w1_ref[...], preferred_element_type=_f32,
                                precision=_HI) + b1_ref[...], 0.0)
        z = jnp.maximum(jnp.dot(z, w2_ref[...], preferred_element_type=_f32,
                                precision=_HI) + b2_ref[...], 0.0)
        out_ref[...] = jnp.dot(z, w3_ref[...], preferred_element_type=_f32,
                               precision=_HI) + b3_ref[...]

    return pl.pallas_call(
        body, out_shape=jax.ShapeDtypeStruct((_G, 2), _f32),
    )(sums, cnt, xmax, wf1, bf1, wf2, bf2, wf3, bf3)


def kernel(x, edge_index, batch, Wc1, bc1, g1, be1, Wc2, bc2, g2, be2,
           Wc3, bc3, g3, be3, Wf1, bf1, Wf2, bf2, Wf3, bf3):
    ei = edge_index.astype(jnp.int32)
    padi = jnp.full((_EP - _E,), _NP - 1, jnp.int32)
    rowp = jnp.concatenate([ei[0], padi]).reshape(_NW, _CH, _K)
    colp = jnp.concatenate([ei[1], padi]).reshape(_NW, _CH, _K)

    zeros8 = jnp.zeros((_NP, _DW), _f32)
    ones8 = jnp.ones((_K, _DW), _f32)
    zerosnd = jnp.zeros((_NP, _D), _f32)

    deg8 = _sc_degree(colp, ones8, zeros8)
    dis, u1 = _tc_prep(deg8, x, Wc1)

    acc1 = _sc_scatter(u1, rowp, colp, zerosnd)
    u2 = _tc_post(acc1, u1, dis, bc1.reshape(1, _D), g1.reshape(1, _D),
                  be1.reshape(1, _D), Wc2)
    acc2 = _sc_scatter(u2, rowp, colp, zerosnd)
    u3 = _tc_post(acc2, u2, dis, bc2.reshape(1, _D), g2.reshape(1, _D),
                  be2.reshape(1, _D), Wc3)
    acc3 = _sc_scatter(u3, rowp, colp, zerosnd)
    h3 = _tc_post_last(acc3, u3, dis, bc3.reshape(1, _D), g3.reshape(1, _D),
                       be3.reshape(1, _D))

    batchp = jnp.concatenate(
        [batch.astype(jnp.int32), jnp.full((_NP - _N,), _G, jnp.int32)]
    ).reshape(1, _NP)
    sums, cnt = _tc_pool_sums(h3, batchp)
    c = cnt[:, 0]
    end = jnp.cumsum(c).astype(jnp.int32)
    start = end - c.astype(jnp.int32)
    xmax = _tc_pool_max(start, end, h3)

    return _tc_mlp(sums, cnt, xmax, Wf1, bf1.reshape(1, _D), Wf2,
                   bf2.reshape(1, _D // 2), Wf3, bf3.reshape(1, 2))


# speedup vs baseline: 1.4574x; 1.0048x over previous
import functools

import jax
import jax.numpy as jnp
from jax import lax
from jax.experimental import pallas as pl
from jax.experimental.pallas import tpu as pltpu
from jax.experimental.pallas import tpu_sc as plsc

_N, _E, _D, _G = 10000, 320000, 128, 256
_NC, _NS = 2, 16
_NW = _NC * _NS
_K = 128
_CH = 80
_EP = _NW * _CH * _K
_CH0 = 40
_CH1 = 120
_TOTCH = _NS * (_CH0 + _CH1)
_EPS2 = _TOTCH * _K
_C0TOT = _NS * _CH0
_NP = 10240
_RT = _NP // _NS
_RB = 2048
_RM = 1024
_EPS = 1e-5
_DW = 128

_f32 = jnp.float32
_HI = lax.Precision.HIGHEST


def _sc_mesh():
    return plsc.VectorSubcoreMesh(
        core_axis_name="c", subcore_axis_name="s",
        num_cores=_NC, num_subcores=_NS)


@functools.cache
def _build_sc_degree():
    return functools.partial(
        pl.kernel,
        out_type=jax.ShapeDtypeStruct((_NC, _NP, _DW), _f32),
        mesh=_sc_mesh(),
        scratch_types=[
            pltpu.VMEM((_CH, _K), jnp.int32),
            pltpu.VMEM((_K, _DW), _f32),
            pltpu.VMEM_SHARED((_NP, _DW), _f32),
        ],
    )(_sc_degree_body)


def _sc_degree_body(col_hbm, ones_hbm, zeros8_hbm, out_hbm, col_v, ones_v, acc):
    cid = lax.axis_index("c")
    sid = lax.axis_index("s")
    w = cid * _NS + sid
    pltpu.sync_copy(zeros8_hbm.at[pl.ds(sid * _RT, _RT)],
                    acc.at[pl.ds(sid * _RT, _RT)])
    pltpu.sync_copy(col_hbm.at[w], col_v)
    pltpu.sync_copy(ones_hbm, ones_v)
    plsc.subcore_barrier()

    def chunk(j, carry):
        pltpu.sync_copy(ones_v, acc.at[col_v.at[j]], add=True)
        return carry

    lax.fori_loop(0, _CH, chunk, 0)
    plsc.subcore_barrier()
    pltpu.sync_copy(acc.at[pl.ds(sid * _RT, _RT)],
                    out_hbm.at[cid, pl.ds(sid * _RT, _RT)])


def _sc_degree(colp, ones8, zeros8):
    return _build_sc_degree()(colp, ones8, zeros8)


@functools.cache
def _build_sc_scatter():
    return functools.partial(
        pl.kernel,
        out_type=jax.ShapeDtypeStruct((_NC, _NP, _D), _f32),
        mesh=_sc_mesh(),
        scratch_types=[
            pltpu.VMEM((2, _K), jnp.int32),
            pltpu.VMEM((_CH1, _K), jnp.int32),
            pltpu.VMEM((_K, _D), _f32),
            pltpu.VMEM((_K, _D), _f32),
            pltpu.VMEM_SHARED((_NP, _D), _f32),
            pltpu.SemaphoreType.DMA,
            pltpu.SemaphoreType.DMA,
            pltpu.SemaphoreType.DMA,
            pltpu.SemaphoreType.DMA,
        ],
    )(_sc_scatter_body)


_NSUB = 4
_KS = _K // _NSUB


def _sc_scatter_body(u_hbm, row_hbm, col_hbm, zeros_hbm, out_hbm,
                     rv, col_v, buf0, buf1, acc, sg0, sg1, sr0, sr1):
    cid = lax.axis_index("c")
    sid = lax.axis_index("s")
    base = jnp.where(cid == 0, sid * _CH0, _C0TOT + sid * _CH1)
    nch = jnp.where(cid == 0, _CH0, _CH1)
    pltpu.sync_copy(zeros_hbm.at[pl.ds(sid * _RT, _RT)],
                    acc.at[pl.ds(sid * _RT, _RT)])
    pltpu.sync_copy(col_hbm.at[pl.ds(base, _CH1)], col_v)
    bufs = (buf0, buf1)
    sgs = (sg0, sg1)
    srs = (sr0, sr1)

    def gathers(slot, buf, sg):
        for q in range(_NSUB):
            pltpu.async_copy(u_hbm.at[rv.at[slot, pl.ds(q * _KS, _KS)]],
                             buf.at[pl.ds(q * _KS, _KS)], sg)

    def drain(slot, buf, sg):
        for q in range(_NSUB):
            pltpu.make_async_copy(
                u_hbm.at[rv.at[slot, pl.ds(q * _KS, _KS)]],
                buf.at[pl.ds(q * _KS, _KS)], sg).wait()

    pltpu.sync_copy(row_hbm.at[base], rv.at[0])
    plsc.subcore_barrier()
    gathers(0, buf0, sg0)
    pltpu.async_copy(row_hbm.at[base + 1], rv.at[1], sr1)

    def step(j2, carry):
        for b in range(2):
            j = j2 * 2 + b
            drain(b, bufs[b], sgs[b])

            @pl.when(j + 1 < nch)
            def _():
                pltpu.make_async_copy(row_hbm.at[base + j + 1], rv.at[1 - b],
                                      srs[1 - b]).wait()
                gathers(1 - b, bufs[1 - b], sgs[1 - b])

            @pl.when(j + 2 < nch)
            def _():
                pltpu.async_copy(row_hbm.at[base + j + 2], rv.at[b], srs[b])

            pltpu.sync_copy(bufs[b], acc.at[col_v.at[j]], add=True)
        return carry

    lax.fori_loop(0, lax.div(nch, 2), step, 0)
    plsc.subcore_barrier()
    pltpu.sync_copy(acc.at[pl.ds(sid * _RT, _RT)],
                    out_hbm.at[cid, pl.ds(sid * _RT, _RT)])


def _sc_scatter(u, rowp, colp, zerosnd):
    return _build_sc_scatter()(u, rowp, colp, zerosnd)


def _tc_prep(deg8, x, w1):
    def body(deg8_ref, x_ref, w_ref, dis_ref, u_ref):
        d8 = deg8_ref[0] + deg8_ref[1]
        deg = d8[:, 0:1] + 1.0
        dis = lax.rsqrt(deg)
        dis_ref[...] = dis
        lin = jnp.dot(x_ref[...], w_ref[...],
                      preferred_element_type=_f32, precision=_HI)
        u_ref[pl.ds(0, _N), :] = dis[:_N] * lin
        u_ref[pl.ds(_N, _NP - _N), :] = jnp.zeros((_NP - _N, _D), _f32)

    return pl.pallas_call(
        body,
        out_shape=[jax.ShapeDtypeStruct((_NP, 1), _f32),
                   jax.ShapeDtypeStruct((_NP, _D), _f32)],
    )(deg8, x, w1)


def _bn_relu(acc_ref, u_ref, dis_ref, b_ref, g_ref, be_ref):
    a = acc_ref[...]
    s = a[0, :_N] + a[1, :_N] + u_ref[pl.ds(0, _N), :]
    dis = dis_ref[...][:_N]
    pre = dis * s + b_ref[...]
    mean = jnp.mean(pre, axis=0, keepdims=True)
    xc = pre - mean
    var = jnp.mean(xc * xc, axis=0, keepdims=True)
    return jnp.maximum(xc * lax.rsqrt(var + _EPS) * g_ref[...] + be_ref[...],
                       0.0)


def _tc_post(acc, u, dis, b, g, be, wn):
    def body(acc_ref, u_ref, dis_ref, b_ref, g_ref, be_ref, w_ref, out_ref):
        h = _bn_relu(acc_ref, u_ref, dis_ref, b_ref, g_ref, be_ref)
        nxt = jnp.dot(h, w_ref[...], preferred_element_type=_f32,
                      precision=_HI)
        out_ref[pl.ds(0, _N), :] = dis_ref[...][:_N] * nxt
        out_ref[pl.ds(_N, _NP - _N), :] = jnp.zeros((_NP - _N, _D), _f32)

    return pl.pallas_call(
        body, out_shape=jax.ShapeDtypeStruct((_NP, _D), _f32),
    )(acc, u, dis, b, g, be, wn)


def _tc_post_last(acc, u, dis, b, g, be):
    def body(acc_ref, u_ref, dis_ref, b_ref, g_ref, be_ref, out_ref):
        h = _bn_relu(acc_ref, u_ref, dis_ref, b_ref, g_ref, be_ref)
        out_ref[pl.ds(0, _N), :] = h
        out_ref[pl.ds(_N, _NP - _N), :] = jnp.zeros((_NP - _N, _D), _f32)

    return pl.pallas_call(
        body, out_shape=jax.ShapeDtypeStruct((_NP, _D), _f32),
    )(acc, u, dis, b, g, be)


def _tc_pool_sums(h3p, batchp):
    nb = _NP // _RB

    def body(h_ref, b_ref, sums_ref, cnt_ref):
        i = pl.program_id(0)
        seg = lax.broadcasted_iota(jnp.int32, (_G, _RB), 0)
        oh = (b_ref[...] == seg).astype(_f32)
        ps = jnp.dot(oh, h_ref[...], preferred_element_type=_f32,
                     precision=_HI)
        pc = jnp.broadcast_to(jnp.sum(oh, axis=1, keepdims=True), (_G, _D))

        @pl.when(i == 0)
        def _():
            sums_ref[...] = ps
            cnt_ref[...] = pc

        @pl.when(i > 0)
        def _():
            sums_ref[...] = sums_ref[...] + ps
            cnt_ref[...] = cnt_ref[...] + pc

    return pl.pallas_call(
        body,
        grid=(nb,),
        in_specs=[pl.BlockSpec((_RB, _D), lambda i: (i, 0)),
                  pl.BlockSpec((1, _RB), lambda i: (0, i))],
        out_specs=[pl.BlockSpec((_G, _D), lambda i: (0, 0)),
                   pl.BlockSpec((_G, _D), lambda i: (0, 0))],
        out_shape=[jax.ShapeDtypeStruct((_G, _D), _f32),
                   jax.ShapeDtypeStruct((_G, _D), _f32)],
    )(h3p, batchp)


def _tc_pool_max(start, end, h3p):
    def body(start_ref, end_ref, h_ref, out_ref):
        g = pl.program_id(0)
        s = start_ref[g]
        e = end_ref[g]
        b0 = lax.div(s, _RM)
        b1 = lax.div(e + (_RM - 1), _RM)

        def blk(b, acc):
            rows = lax.broadcasted_iota(jnp.int32, (_RM, 1), 0) + b * _RM
            m = jnp.logical_and(rows >= s, rows < e)
            vals = jnp.where(m, h_ref[pl.ds(b * _RM, _RM), :], 0.0)
            return jnp.maximum(acc, jnp.max(vals, axis=0, keepdims=True))

        acc = lax.fori_loop(b0, b1, blk, jnp.zeros((1, _D), _f32))
        out_ref[...] = acc.reshape(1, 1, _D)

    return pl.pallas_call(
        body,
        grid=(_G,),
        in_specs=[pl.BlockSpec(memory_space=pltpu.SMEM),
                  pl.BlockSpec(memory_space=pltpu.SMEM),
                  pl.BlockSpec(memory_space=pltpu.VMEM)],
        out_specs=pl.BlockSpec((1, 1, _D), lambda g: (g, 0, 0)),
        out_shape=jax.ShapeDtypeStruct((_G, 1, _D), _f32),
    )(start, end, h3p)


def _tc_mlp(sums, cnt, xmax, wf1, bf1, wf2, bf2, wf3, bf3):
    def body(sums_ref, cnt_ref, xmax_ref, w1_ref, b1_ref, w2_ref, b2_ref,
             w3_ref, b3_ref, out_ref):
        counts = cnt_ref[...][:, 0:1]
        mean = sums_ref[...] / jnp.maximum(counts, 1.0)
        z = jnp.concatenate([mean, xmax_ref[...].reshape(_G, _D)], axis=1)
        z = jnp.maximum(jnp.dot(z, w1_ref[...], preferred_element_type=_f32,
                                precision=_HI) + b1_ref[...], 0.0)
        z = jnp.maximum(jnp.dot(z, w2_ref[...], preferred_element_type=_f32,
                                precision=_HI) + b2_ref[...], 0.0)
        out_ref[...] = jnp.dot(z, w3_ref[...], preferred_element_type=_f32,
                               precision=_HI) + b3_ref[...]

    return pl.pallas_call(
        body, out_shape=jax.ShapeDtypeStruct((_G, 2), _f32),
    )(sums, cnt, xmax, wf1, bf1, wf2, bf2, wf3, bf3)


def kernel(x, edge_index, batch, Wc1, bc1, g1, be1, Wc2, bc2, g2, be2,
           Wc3, bc3, g3, be3, Wf1, bf1, Wf2, bf2, Wf3, bf3):
    ei = edge_index.astype(jnp.int32)
    padi = jnp.full((_EP - _E,), _NP - 1, jnp.int32)
    colp = jnp.concatenate([ei[1], padi]).reshape(_NW, _CH, _K)
    padi2 = jnp.full((_EPS2 - _E,), _NP - 1, jnp.int32)
    rowp2 = jnp.concatenate([ei[0], padi2]).reshape(_TOTCH, _K)
    colp2 = jnp.concatenate([ei[1], padi2]).reshape(_TOTCH, _K)

    zeros8 = jnp.zeros((_NP, _DW), _f32)
    ones8 = jnp.ones((_K, _DW), _f32)
    zerosnd = jnp.zeros((_NP, _D), _f32)

    deg8 = _sc_degree(colp, ones8, zeros8)
    dis, u1 = _tc_prep(deg8, x, Wc1)

    acc1 = _sc_scatter(u1, rowp2, colp2, zerosnd)
    u2 = _tc_post(acc1, u1, dis, bc1.reshape(1, _D), g1.reshape(1, _D),
                  be1.reshape(1, _D), Wc2)
    acc2 = _sc_scatter(u2, rowp2, colp2, zerosnd)
    u3 = _tc_post(acc2, u2, dis, bc2.reshape(1, _D), g2.reshape(1, _D),
                  be2.reshape(1, _D), Wc3)
    acc3 = _sc_scatter(u3, rowp2, colp2, zerosnd)
    h3 = _tc_post_last(acc3, u3, dis, bc3.reshape(1, _D), g3.reshape(1, _D),
                       be3.reshape(1, _D))

    batchp = jnp.concatenate(
        [batch.astype(jnp.int32), jnp.full((_NP - _N,), _G, jnp.int32)]
    ).reshape(1, _NP)
    sums, cnt = _tc_pool_sums(h3, batchp)
    c = cnt[:, 0]
    end = jnp.cumsum(c).astype(jnp.int32)
    start = end - c.astype(jnp.int32)
    xmax = _tc_pool_max(start, end, h3)

    return _tc_mlp(sums, cnt, xmax, Wf1, bf1.reshape(1, _D), Wf2,
                   bf2.reshape(1, _D // 2), Wf3, bf3.reshape(1, 2))


# speedup vs baseline: 1.4643x; 1.0047x over previous
import functools

import jax
import jax.numpy as jnp
from jax import lax
from jax.experimental import pallas as pl
from jax.experimental.pallas import tpu as pltpu
from jax.experimental.pallas import tpu_sc as plsc

_N, _E, _D, _G = 10000, 320000, 128, 256
_NC, _NS = 2, 16
_NW = _NC * _NS
_K = 128
_CH = 80
_EP = _NW * _CH * _K
_CH0 = 40
_CH1 = 120
_TOTCH = _NS * (_CH0 + _CH1)
_EPS2 = _TOTCH * _K
_C0TOT = _NS * _CH0
_NP = 10240
_RT = _NP // _NS
_RB = 2048
_RM = 1024
_EPS = 1e-5
_DW = 128

_f32 = jnp.float32
_HI = lax.Precision.HIGHEST


def _sc_mesh():
    return plsc.VectorSubcoreMesh(
        core_axis_name="c", subcore_axis_name="s",
        num_cores=_NC, num_subcores=_NS)


@functools.cache
def _build_sc_degree():
    return functools.partial(
        pl.kernel,
        out_type=jax.ShapeDtypeStruct((_NC, _NP, _DW), _f32),
        mesh=_sc_mesh(),
        scratch_types=[
            pltpu.VMEM((_CH, _K), jnp.int32),
            pltpu.VMEM((_K, _DW), _f32),
            pltpu.VMEM_SHARED((_NP, _DW), _f32),
        ],
    )(_sc_degree_body)


def _sc_degree_body(col_hbm, ones_hbm, zeros8_hbm, out_hbm, col_v, ones_v, acc):
    cid = lax.axis_index("c")
    sid = lax.axis_index("s")
    w = cid * _NS + sid
    pltpu.sync_copy(zeros8_hbm.at[pl.ds(sid * _RT, _RT)],
                    acc.at[pl.ds(sid * _RT, _RT)])
    pltpu.sync_copy(col_hbm.at[w], col_v)
    pltpu.sync_copy(ones_hbm, ones_v)
    plsc.subcore_barrier()

    def chunk(j, carry):
        pltpu.sync_copy(ones_v, acc.at[col_v.at[j]], add=True)
        return carry

    lax.fori_loop(0, _CH, chunk, 0)
    plsc.subcore_barrier()
    pltpu.sync_copy(acc.at[pl.ds(sid * _RT, _RT)],
                    out_hbm.at[cid, pl.ds(sid * _RT, _RT)])


def _sc_degree(colp, ones8, zeros8):
    return _build_sc_degree()(colp, ones8, zeros8)


@functools.cache
def _build_sc_scatter():
    return functools.partial(
        pl.kernel,
        out_type=jax.ShapeDtypeStruct((_NC, _NP, _D), _f32),
        mesh=_sc_mesh(),
        scratch_types=[
            pltpu.VMEM((2, _K), jnp.int32),
            pltpu.VMEM((_CH1, _K), jnp.int32),
            pltpu.VMEM((_K, _D), _f32),
            pltpu.VMEM((_K, _D), _f32),
            pltpu.VMEM_SHARED((_NP, _D), _f32),
            pltpu.SemaphoreType.DMA,
            pltpu.SemaphoreType.DMA,
            pltpu.SemaphoreType.DMA,
            pltpu.SemaphoreType.DMA,
        ],
    )(_sc_scatter_body)


_NSUB = 4
_KS = _K // _NSUB


def _sc_scatter_body(u_hbm, row_hbm, col_hbm, zeros_hbm, out_hbm,
                     rv, col_v, buf0, buf1, acc, sg0, sg1, sr0, sr1):
    cid = lax.axis_index("c")
    sid = lax.axis_index("s")
    base = jnp.where(cid == 0, sid * _CH0, _C0TOT + sid * _CH1)
    nch = jnp.where(cid == 0, _CH0, _CH1)
    pltpu.sync_copy(zeros_hbm.at[pl.ds(sid * _RT, _RT)],
                    acc.at[pl.ds(sid * _RT, _RT)])
    pltpu.sync_copy(col_hbm.at[pl.ds(base, _CH1)], col_v)
    bufs = (buf0, buf1)
    sgs = (sg0, sg1)
    srs = (sr0, sr1)

    def gathers(slot, buf, sg):
        for q in range(_NSUB):
            pltpu.async_copy(u_hbm.at[rv.at[slot, pl.ds(q * _KS, _KS)]],
                             buf.at[pl.ds(q * _KS, _KS)], sg)

    def drain(slot, buf, sg):
        for q in range(_NSUB):
            pltpu.make_async_copy(
                u_hbm.at[rv.at[slot, pl.ds(q * _KS, _KS)]],
                buf.at[pl.ds(q * _KS, _KS)], sg).wait()

    pltpu.sync_copy(row_hbm.at[base], rv.at[0])
    plsc.subcore_barrier()
    gathers(0, buf0, sg0)
    pltpu.async_copy(row_hbm.at[base + 1], rv.at[1], sr1)

    def step(j2, carry):
        for b in range(2):
            j = j2 * 2 + b
            drain(b, bufs[b], sgs[b])

            @pl.when(j + 1 < nch)
            def _():
                pltpu.make_async_copy(row_hbm.at[base + j + 1], rv.at[1 - b],
                                      srs[1 - b]).wait()
                gathers(1 - b, bufs[1 - b], sgs[1 - b])

            @pl.when(j + 2 < nch)
            def _():
                pltpu.async_copy(row_hbm.at[base + j + 2], rv.at[b], srs[b])

            pltpu.sync_copy(bufs[b], acc.at[col_v.at[j]], add=True)
        return carry

    lax.fori_loop(0, lax.div(nch, 2), step, 0)
    plsc.subcore_barrier()
    pltpu.sync_copy(acc.at[pl.ds(sid * _RT, _RT)],
                    out_hbm.at[cid, pl.ds(sid * _RT, _RT)])


def _sc_scatter(u, rowp, colp, zerosnd):
    return _build_sc_scatter()(u, rowp, colp, zerosnd)


def _tc_prep(deg8, x, w1):
    def body(deg8_ref, x_ref, w_ref, dis_ref, u_ref):
        d8 = deg8_ref[0] + deg8_ref[1]
        deg = d8[:, 0:1] + 1.0
        dis = 1.0 / jnp.sqrt(deg)
        dis_ref[...] = dis
        lin = jnp.dot(x_ref[...], w_ref[...],
                      preferred_element_type=_f32)
        u_ref[pl.ds(0, _N), :] = dis[:_N] * lin
        u_ref[pl.ds(_N, _NP - _N), :] = jnp.zeros((_NP - _N, _D), _f32)

    return pl.pallas_call(
        body,
        out_shape=[jax.ShapeDtypeStruct((_NP, 1), _f32),
                   jax.ShapeDtypeStruct((_NP, _D), _f32)],
    )(deg8, x, w1)


def _bn_relu(acc_ref, u_ref, dis_ref, b_ref, g_ref, be_ref):
    a = acc_ref[...]
    s = a[0, :_N] + a[1, :_N] + u_ref[pl.ds(0, _N), :]
    dis = dis_ref[...][:_N]
    pre = dis * s + b_ref[...]
    mean = jnp.mean(pre, axis=0, keepdims=True)
    xc = pre - mean
    var = jnp.mean(xc * xc, axis=0, keepdims=True)
    return jnp.maximum(xc / jnp.sqrt(var + _EPS) * g_ref[...] + be_ref[...],
                       0.0)


def _tc_post(acc, u, dis, b, g, be, wn):
    def body(acc_ref, u_ref, dis_ref, b_ref, g_ref, be_ref, w_ref, out_ref):
        h = _bn_relu(acc_ref, u_ref, dis_ref, b_ref, g_ref, be_ref)
        nxt = jnp.dot(h, w_ref[...], preferred_element_type=_f32)
        out_ref[pl.ds(0, _N), :] = dis_ref[...][:_N] * nxt
        out_ref[pl.ds(_N, _NP - _N), :] = jnp.zeros((_NP - _N, _D), _f32)

    return pl.pallas_call(
        body, out_shape=jax.ShapeDtypeStruct((_NP, _D), _f32),
    )(acc, u, dis, b, g, be, wn)


def _tc_post_last(acc, u, dis, b, g, be):
    def body(acc_ref, u_ref, dis_ref, b_ref, g_ref, be_ref, out_ref):
        h = _bn_relu(acc_ref, u_ref, dis_ref, b_ref, g_ref, be_ref)
        out_ref[pl.ds(0, _N), :] = h
        out_ref[pl.ds(_N, _NP - _N), :] = jnp.zeros((_NP - _N, _D), _f32)

    return pl.pallas_call(
        body, out_shape=jax.ShapeDtypeStruct((_NP, _D), _f32),
    )(acc, u, dis, b, g, be)


def _tc_pool_sums(h3p, batchp):
    nb = _NP // _RB

    def body(h_ref, b_ref, sums_ref, cnt_ref):
        i = pl.program_id(0)
        seg = lax.broadcasted_iota(jnp.int32, (_G, _RB), 0)
        oh = (b_ref[...] == seg).astype(_f32)
        ps = jnp.dot(oh, h_ref[...], preferred_element_type=_f32,
                     precision=_HI)
        pc = jnp.broadcast_to(jnp.sum(oh, axis=1, keepdims=True), (_G, _D))

        @pl.when(i == 0)
        def _():
            sums_ref[...] = ps
            cnt_ref[...] = pc

        @pl.when(i > 0)
        def _():
            sums_ref[...] = sums_ref[...] + ps
            cnt_ref[...] = cnt_ref[...] + pc

    return pl.pallas_call(
        body,
        grid=(nb,),
        in_specs=[pl.BlockSpec((_RB, _D), lambda i: (i, 0)),
                  pl.BlockSpec((1, _RB), lambda i: (0, i))],
        out_specs=[pl.BlockSpec((_G, _D), lambda i: (0, 0)),
                   pl.BlockSpec((_G, _D), lambda i: (0, 0))],
        out_shape=[jax.ShapeDtypeStruct((_G, _D), _f32),
                   jax.ShapeDtypeStruct((_G, _D), _f32)],
    )(h3p, batchp)


def _tc_pool_max(start, end, h3p):
    def body(start_ref, end_ref, h_ref, out_ref):
        g = pl.program_id(0)
        s = start_ref[g]
        e = end_ref[g]
        b0 = lax.div(s, _RM)
        b1 = lax.div(e + (_RM - 1), _RM)

        def blk(b, acc):
            rows = lax.broadcasted_iota(jnp.int32, (_RM, 1), 0) + b * _RM
            m = jnp.logical_and(rows >= s, rows < e)
            vals = jnp.where(m, h_ref[pl.ds(b * _RM, _RM), :], 0.0)
            return jnp.maximum(acc, jnp.max(vals, axis=0, keepdims=True))

        acc = lax.fori_loop(b0, b1, blk, jnp.zeros((1, _D), _f32))
        out_ref[...] = acc.reshape(1, 1, _D)

    return pl.pallas_call(
        body,
        grid=(_G,),
        in_specs=[pl.BlockSpec(memory_space=pltpu.SMEM),
                  pl.BlockSpec(memory_space=pltpu.SMEM),
                  pl.BlockSpec(memory_space=pltpu.VMEM)],
        out_specs=pl.BlockSpec((1, 1, _D), lambda g: (g, 0, 0)),
        out_shape=jax.ShapeDtypeStruct((_G, 1, _D), _f32),
    )(start, end, h3p)


def _tc_mlp(sums, cnt, xmax, wf1, bf1, wf2, bf2, wf3, bf3):
    def body(sums_ref, cnt_ref, xmax_ref, w1_ref, b1_ref, w2_ref, b2_ref,
             w3_ref, b3_ref, out_ref):
        counts = cnt_ref[...][:, 0:1]
        mean = sums_ref[...] / jnp.maximum(counts, 1.0)
        z = jnp.concatenate([mean, xmax_ref[...].reshape(_G, _D)], axis=1)
        z = jnp.maximum(jnp.dot(z, w1_ref[...], preferred_element_type=_f32)
                        + b1_ref[...], 0.0)
        z = jnp.maximum(jnp.dot(z, w2_ref[...], preferred_element_type=_f32)
                        + b2_ref[...], 0.0)
        out_ref[...] = jnp.dot(z, w3_ref[...], preferred_element_type=_f32) \
                       + b3_ref[...]

    return pl.pallas_call(
        body, out_shape=jax.ShapeDtypeStruct((_G, 2), _f32),
    )(sums, cnt, xmax, wf1, bf1, wf2, bf2, wf3, bf3)


def kernel(x, edge_index, batch, Wc1, bc1, g1, be1, Wc2, bc2, g2, be2,
           Wc3, bc3, g3, be3, Wf1, bf1, Wf2, bf2, Wf3, bf3):
    ei = edge_index.astype(jnp.int32)
    padi = jnp.full((_EP - _E,), _NP - 1, jnp.int32)
    colp = jnp.concatenate([ei[1], padi]).reshape(_NW, _CH, _K)
    padi2 = jnp.full((_EPS2 - _E,), _NP - 1, jnp.int32)
    rowp2 = jnp.concatenate([ei[0], padi2]).reshape(_TOTCH, _K)
    colp2 = jnp.concatenate([ei[1], padi2]).reshape(_TOTCH, _K)

    zeros8 = jnp.zeros((_NP, _DW), _f32)
    ones8 = jnp.ones((_K, _DW), _f32)
    zerosnd = jnp.zeros((_NP, _D), _f32)

    deg8 = _sc_degree(colp, ones8, zeros8)
    dis, u1 = _tc_prep(deg8, x, Wc1)

    acc1 = _sc_scatter(u1, rowp2, colp2, zerosnd)
    u2 = _tc_post(acc1, u1, dis, bc1.reshape(1, _D), g1.reshape(1, _D),
                  be1.reshape(1, _D), Wc2)
    acc2 = _sc_scatter(u2, rowp2, colp2, zerosnd)
    u3 = _tc_post(acc2, u2, dis, bc2.reshape(1, _D), g2.reshape(1, _D),
                  be2.reshape(1, _D), Wc3)
    acc3 = _sc_scatter(u3, rowp2, colp2, zerosnd)
    h3 = _tc_post_last(acc3, u3, dis, bc3.reshape(1, _D), g3.reshape(1, _D),
                       be3.reshape(1, _D))

    batchp = jnp.concatenate(
        [batch.astype(jnp.int32), jnp.full((_NP - _N,), _G, jnp.int32)]
    ).reshape(1, _NP)
    sums, cnt = _tc_pool_sums(h3, batchp)
    c = cnt[:, 0]
    end = jnp.cumsum(c).astype(jnp.int32)
    start = end - c.astype(jnp.int32)
    xmax = _tc_pool_max(start, end, h3)

    return _tc_mlp(sums, cnt, xmax, Wf1, bf1.reshape(1, _D), Wf2,
                   bf2.reshape(1, _D // 2), Wf3, bf3.reshape(1, 2))
